# baseline (speedup 1.0000x reference)
import functools

import jax
import jax.numpy as jnp
from jax import lax
from jax.experimental import pallas as pl
from jax.experimental.pallas import tpu as pltpu
from jax.experimental.pallas import tpu_sc as plsc

_S = 1024
_A = 4
_ROWS = 256
_PAD = 16
_MLP_LANES = 512


def _leaky(x):
    return jnp.where(x >= 0, x, 0.2 * x)


def _top4(d, n_cols):
    iota = lax.broadcasted_iota(jnp.int32, d.shape, 1)
    vals, idxs = [], []
    for _ in range(4):
        m = jnp.min(d, axis=1, keepdims=True)
        idx = jnp.min(jnp.where(d == m, iota, n_cols), axis=1, keepdims=True)
        d = jnp.where(iota == idx, jnp.inf, d)
        vals.append(m)
        idxs.append(idx)
    return jnp.concatenate(vals, axis=1), jnp.concatenate(idxs, axis=1)


def _knn_body(xs_ref, xt_ref, xst_ref, vals_ref, idx_ref):
    b = pl.program_id(0)
    xsb = xs_ref[0]
    xb = xt_ref[0]
    xsall = xst_ref[0]
    n = xb.shape[1]
    s = xsall.shape[1]
    sqs = jnp.sum(xsb * xsb, axis=1, keepdims=True)
    sqn = jnp.sum(xb * xb, axis=0, keepdims=True)
    sqss = jnp.sum(xsall * xsall, axis=0, keepdims=True)
    dims = (((1,), (0,)), ((), ()))
    dotb = lax.dot_general(xsb, xb, dims,
                           preferred_element_type=jnp.float32)
    dota = lax.dot_general(xsb, xsall, dims,
                           preferred_element_type=jnp.float32)
    db = jnp.sqrt(jnp.maximum((sqs + sqn) - 2.0 * dotb, 0.0) + 1e-8)
    da = jnp.sqrt(jnp.maximum((sqs + sqss) - 2.0 * dota, 0.0) + 1e-8)
    vb, ib = _top4(db, n)
    va, ia = _top4(da, s)
    base = b * n
    vals_ref[0, :, :] = jnp.concatenate([vb, va], axis=1)
    gi = jnp.concatenate([ib + base, ia * (n // s) + base], axis=1)
    idx_ref[...] = jnp.transpose(gi)


def _mlp_body(g3_ref, nd_ref, f_ref, w1_ref, b1_ref, w2_ref,
              b2_ref, w3a_ref, w3b_ref, b3_ref, out_ref):
    nd = nd_ref[...]
    coords = [g3_ref[p, :, 0:3] for p in range(8)]
    sq = [jnp.sum(c * c, axis=1, keepdims=True) for c in coords]
    L = nd.shape[0]
    C = min(L, 256)
    eye = (lax.broadcasted_iota(jnp.int32, (C, C), 0)
           == lax.broadcasted_iota(jnp.int32, (C, C), 1)).astype(jnp.float32)
    dims_t = (((1,), (1,)), ((), ()))
    dims = (((1,), (0,)), ((), ()))

    def dist(u, v):
        parts = []
        for c0 in range(0, L, C):
            mm = lax.dot_general(coords[u][c0:c0 + C], coords[v][c0:c0 + C],
                                 dims_t, preferred_element_type=jnp.float32)
            parts.append(jnp.sum(mm * eye, axis=1, keepdims=True))
        duv = jnp.concatenate(parts, axis=0)
        return jnp.sqrt(jnp.maximum((sq[u] + sq[v]) - 2.0 * duv, 0.0) + 1e-8)

    cols = [nd[:, 1:2], nd[:, 2:3], nd[:, 3:4],
            dist(1, 2), dist(1, 3), dist(2, 3),
            nd[:, 5:6], nd[:, 6:7], nd[:, 7:8],
            dist(5, 6), dist(5, 7), dist(6, 7)]
    for i in range(4):
        for j in range(4):
            cols.append(dist(i, 4 + j))
    fea = jnp.concatenate(cols, axis=1)

    h = _leaky(lax.dot_general(fea, w1_ref[...], dims,
                               preferred_element_type=jnp.float32)
               + b1_ref[...])
    h = _leaky(lax.dot_general(h, w2_ref[...], dims,
                               preferred_element_type=jnp.float32)
               + b2_ref[...])
    o = (lax.dot_general(f_ref[...], w3a_ref[...], dims,
                         preferred_element_type=jnp.float32)
         + lax.dot_general(h, w3b_ref[...], dims,
                           preferred_element_type=jnp.float32)
         + b3_ref[...])
    out_ref[...] = _leaky(o)


def _build_gather(n_rows, width):
    info = plsc.get_sparse_core_info()
    nw = info.num_cores * info.num_subcores
    b_per_w = n_rows // nw
    mesh = plsc.VectorSubcoreMesh(core_axis_name="c", subcore_axis_name="s")

    @functools.partial(
        pl.kernel, mesh=mesh,
        compiler_params=pltpu.CompilerParams(use_tc_tiling_on_sc=False),
        out_type=jax.ShapeDtypeStruct((n_rows, width), jnp.float32),
        scratch_types=[
            pltpu.VMEM((b_per_w,), jnp.int32),
            pltpu.VMEM((b_per_w, width), jnp.float32),
            pltpu.SemaphoreType.DMA,
        ],
    )
    def gather_k(table_hbm, idx_hbm, out_hbm, idx_v, rows_v, sem):
        wid = lax.axis_index("s") * info.num_cores + lax.axis_index("c")
        base = wid * b_per_w
        pltpu.sync_copy(idx_hbm.at[pl.ds(base, b_per_w)], idx_v)
        pltpu.async_copy(table_hbm.at[idx_v], rows_v, sem).wait()
        pltpu.sync_copy(rows_v, out_hbm.at[pl.ds(base, b_per_w)])

    return gather_k


def kernel(xyz, feature, W1, b1, W2, b2, W3, b3):
    B, N, _ = xyz.shape
    S = _S
    stride = N // S
    Cin = feature.shape[2]
    T = W1.shape[1]
    Cout = W3.shape[1]

    xs = xyz[:, ::stride, :]
    xT = jnp.transpose(xyz, (0, 2, 1))
    xsT = jnp.transpose(xs, (0, 2, 1))
    table = jnp.pad(xyz.reshape(B * N, 3), ((0, 0), (0, _PAD - 3)))
    f = feature[:, ::stride, :].reshape(B * S, Cin)

    knn = pl.pallas_call(
        _knn_body,
        grid=(B, S // _ROWS),
        in_specs=[
            pl.BlockSpec((1, _ROWS, 3), lambda b, i: (b, i, 0)),
            pl.BlockSpec((1, 3, N), lambda b, i: (b, 0, 0)),
            pl.BlockSpec((1, 3, S), lambda b, i: (b, 0, 0)),
        ],
        out_specs=[
            pl.BlockSpec((1, _ROWS, 8), lambda b, i: (b, i, 0)),
            pl.BlockSpec((8, _ROWS), lambda b, i: (0, b * (_S // _ROWS) + i)),
        ],
        out_shape=[
            jax.ShapeDtypeStruct((B, S, 8), jnp.float32),
            jax.ShapeDtypeStruct((8, B * S), jnp.int32),
        ],
    )
    vals, idxs = knn(xs, xT, xsT)

    gath = _build_gather(B * S * 8, _PAD)(table, idxs.reshape(B * S * 8))
    g3 = gath.reshape(8, B * S, _PAD)
    nd = vals.reshape(B * S, 8)

    L = _MLP_LANES
    mlp = pl.pallas_call(
        _mlp_body,
        grid=(B * S // L,),
        in_specs=[
            pl.BlockSpec((8, L, _PAD), lambda j: (0, j, 0)),
            pl.BlockSpec((L, 8), lambda j: (j, 0)),
            pl.BlockSpec((L, Cin), lambda j: (j, 0)),
            pl.BlockSpec((W1.shape[0], T), lambda j: (0, 0)),
            pl.BlockSpec((1, T), lambda j: (0, 0)),
            pl.BlockSpec((T, T), lambda j: (0, 0)),
            pl.BlockSpec((1, T), lambda j: (0, 0)),
            pl.BlockSpec((Cin, Cout), lambda j: (0, 0)),
            pl.BlockSpec((T, Cout), lambda j: (0, 0)),
            pl.BlockSpec((1, Cout), lambda j: (0, 0)),
        ],
        out_specs=pl.BlockSpec((L, Cout), lambda j: (j, 0)),
        out_shape=jax.ShapeDtypeStruct((B * S, Cout), jnp.float32),
    )
    out = mlp(g3, nd, f, W1, b1[None, :], W2, b2[None, :],
              W3[:Cin], W3[Cin:], b3[None, :])
    return out.reshape(B, S, Cout)


# speedup vs baseline: 1.0485x; 1.0485x over previous
import functools

import jax
import jax.numpy as jnp
from jax import lax
from jax.experimental import pallas as pl
from jax.experimental.pallas import tpu as pltpu
from jax.experimental.pallas import tpu_sc as plsc

_S = 1024
_A = 4
_ROWS = 256
_PAD = 128
_MLP_LANES = 512


def _leaky(x):
    return jnp.where(x >= 0, x, 0.2 * x)


def _top4(d, n_cols):
    iota = lax.broadcasted_iota(jnp.int32, d.shape, 1)
    vals, idxs = [], []
    for _ in range(4):
        m = jnp.min(d, axis=1, keepdims=True)
        idx = jnp.min(jnp.where(d == m, iota, n_cols), axis=1, keepdims=True)
        d = jnp.where(iota == idx, jnp.inf, d)
        vals.append(m)
        idxs.append(idx)
    return jnp.concatenate(vals, axis=1), jnp.concatenate(idxs, axis=1)


def _knn_body(xs_ref, xt_ref, xst_ref, vals_ref, idx_ref):
    b = pl.program_id(0)
    xsb = xs_ref[0]
    xb = xt_ref[0]
    xsall = xst_ref[0]
    n = xb.shape[1]
    s = xsall.shape[1]
    sqs = jnp.sum(xsb * xsb, axis=1, keepdims=True)
    sqn = jnp.sum(xb * xb, axis=0, keepdims=True)
    sqss = jnp.sum(xsall * xsall, axis=0, keepdims=True)
    dims = (((1,), (0,)), ((), ()))
    dotb = lax.dot_general(xsb, xb, dims,
                           preferred_element_type=jnp.float32)
    dota = lax.dot_general(xsb, xsall, dims,
                           preferred_element_type=jnp.float32)
    db = jnp.sqrt(jnp.maximum((sqs + sqn) - 2.0 * dotb, 0.0) + 1e-8)
    da = jnp.sqrt(jnp.maximum((sqs + sqss) - 2.0 * dota, 0.0) + 1e-8)
    vb, ib = _top4(db, n)
    va, ia = _top4(da, s)
    base = b * n
    vals_ref[0, :, :] = jnp.concatenate([vb, va], axis=1)
    gi = jnp.concatenate([ib + base, ia * (n // s) + base], axis=1)
    idx_ref[...] = jnp.transpose(gi)


def _mlp_body(g3_ref, nd_ref, f_ref, w1_ref, b1_ref, w2_ref,
              b2_ref, w3a_ref, w3b_ref, b3_ref, out_ref):
    nd = nd_ref[...]
    coords = [g3_ref[p, :, 0:3] for p in range(8)]
    sq = [jnp.sum(c * c, axis=1, keepdims=True) for c in coords]
    L = nd.shape[0]
    C = min(L, 256)
    eye = (lax.broadcasted_iota(jnp.int32, (C, C), 0)
           == lax.broadcasted_iota(jnp.int32, (C, C), 1)).astype(jnp.float32)
    dims_t = (((1,), (1,)), ((), ()))
    dims = (((1,), (0,)), ((), ()))

    def dist(u, v):
        parts = []
        for c0 in range(0, L, C):
            mm = lax.dot_general(coords[u][c0:c0 + C], coords[v][c0:c0 + C],
                                 dims_t, preferred_element_type=jnp.float32)
            parts.append(jnp.sum(mm * eye, axis=1, keepdims=True))
        duv = jnp.concatenate(parts, axis=0)
        return jnp.sqrt(jnp.maximum((sq[u] + sq[v]) - 2.0 * duv, 0.0) + 1e-8)

    cols = [nd[:, 1:2], nd[:, 2:3], nd[:, 3:4],
            dist(1, 2), dist(1, 3), dist(2, 3),
            nd[:, 5:6], nd[:, 6:7], nd[:, 7:8],
            dist(5, 6), dist(5, 7), dist(6, 7)]
    for i in range(4):
        for j in range(4):
            cols.append(dist(i, 4 + j))
    fea = jnp.concatenate(cols, axis=1)

    h = _leaky(lax.dot_general(fea, w1_ref[...], dims,
                               preferred_element_type=jnp.float32)
               + b1_ref[...])
    h = _leaky(lax.dot_general(h, w2_ref[...], dims,
                               preferred_element_type=jnp.float32)
               + b2_ref[...])
    o = (lax.dot_general(f_ref[...], w3a_ref[...], dims,
                         preferred_element_type=jnp.float32)
         + lax.dot_general(h, w3b_ref[...], dims,
                           preferred_element_type=jnp.float32)
         + b3_ref[...])
    out_ref[...] = _leaky(o)


def _build_gather(n_rows, width):
    info = plsc.get_sparse_core_info()
    nw = info.num_cores * info.num_subcores
    b_per_w = n_rows // nw
    mesh = plsc.VectorSubcoreMesh(core_axis_name="c", subcore_axis_name="s")

    @functools.partial(
        pl.kernel, mesh=mesh,
        out_type=jax.ShapeDtypeStruct((n_rows, width), jnp.float32),
        scratch_types=[
            pltpu.VMEM((b_per_w,), jnp.int32),
            pltpu.VMEM((b_per_w, width), jnp.float32),
            pltpu.SemaphoreType.DMA,
        ],
    )
    def gather_k(table_hbm, idx_hbm, out_hbm, idx_v, rows_v, sem):
        wid = lax.axis_index("s") * info.num_cores + lax.axis_index("c")
        base = wid * b_per_w
        pltpu.sync_copy(idx_hbm.at[pl.ds(base, b_per_w)], idx_v)
        pltpu.async_copy(table_hbm.at[idx_v], rows_v, sem).wait()
        pltpu.sync_copy(rows_v, out_hbm.at[pl.ds(base, b_per_w)])

    return gather_k


def kernel(xyz, feature, W1, b1, W2, b2, W3, b3):
    B, N, _ = xyz.shape
    S = _S
    stride = N // S
    Cin = feature.shape[2]
    T = W1.shape[1]
    Cout = W3.shape[1]

    xs = xyz[:, ::stride, :]
    xT = jnp.transpose(xyz, (0, 2, 1))
    xsT = jnp.transpose(xs, (0, 2, 1))
    table = jnp.pad(xyz.reshape(B * N, 3), ((0, 0), (0, _PAD - 3)))
    f = feature[:, ::stride, :].reshape(B * S, Cin)

    knn = pl.pallas_call(
        _knn_body,
        grid=(B, S // _ROWS),
        in_specs=[
            pl.BlockSpec((1, _ROWS, 3), lambda b, i: (b, i, 0)),
            pl.BlockSpec((1, 3, N), lambda b, i: (b, 0, 0)),
            pl.BlockSpec((1, 3, S), lambda b, i: (b, 0, 0)),
        ],
        out_specs=[
            pl.BlockSpec((1, _ROWS, 8), lambda b, i: (b, i, 0)),
            pl.BlockSpec((8, _ROWS), lambda b, i: (0, b * (_S // _ROWS) + i)),
        ],
        out_shape=[
            jax.ShapeDtypeStruct((B, S, 8), jnp.float32),
            jax.ShapeDtypeStruct((8, B * S), jnp.int32),
        ],
    )
    vals, idxs = knn(xs, xT, xsT)

    gath = _build_gather(B * S * 8, _PAD)(table, idxs.reshape(B * S * 8))
    g3 = gath.reshape(8, B * S, _PAD)
    nd = vals.reshape(B * S, 8)

    L = _MLP_LANES
    mlp = pl.pallas_call(
        _mlp_body,
        grid=(B * S // L,),
        in_specs=[
            pl.BlockSpec((8, L, _PAD), lambda j: (0, j, 0)),
            pl.BlockSpec((L, 8), lambda j: (j, 0)),
            pl.BlockSpec((L, Cin), lambda j: (j, 0)),
            pl.BlockSpec((W1.shape[0], T), lambda j: (0, 0)),
            pl.BlockSpec((1, T), lambda j: (0, 0)),
            pl.BlockSpec((T, T), lambda j: (0, 0)),
            pl.BlockSpec((1, T), lambda j: (0, 0)),
            pl.BlockSpec((Cin, Cout), lambda j: (0, 0)),
            pl.BlockSpec((T, Cout), lambda j: (0, 0)),
            pl.BlockSpec((1, Cout), lambda j: (0, 0)),
        ],
        out_specs=pl.BlockSpec((L, Cout), lambda j: (j, 0)),
        out_shape=jax.ShapeDtypeStruct((B * S, Cout), jnp.float32),
    )
    out = mlp(g3, nd, f, W1, b1[None, :], W2, b2[None, :],
              W3[:Cin], W3[Cin:], b3[None, :])
    return out.reshape(B, S, Cout)
